# Initial kernel scaffold; baseline (speedup 1.0000x reference)
import jax
import jax.numpy as jnp
from jax.experimental import pallas as pl


def kernel(hidden_states, W):
    raise NotImplementedError("write your pallas kernel here")


# speedup vs baseline: 1.3930x; 1.3930x over previous
import jax
import jax.numpy as jnp
from jax.experimental import pallas as pl

NUM_EXPERTS = 8
TOP_K = 2
HIDDEN = 2048
TOKENS = 2048
ROWS = NUM_EXPERTS * TOKENS

GRID = 16
TBLK = TOKENS // GRID
RBLK = ROWS // GRID


def _body(w_ref, h_ref, scores_ref, idx_ref):
    i = pl.program_id(0)
    lt = jax.lax.dot_general(
        w_ref[...], h_ref[...], (((1,), (1,)), ((), ())),
        preferred_element_type=jnp.float32)
    eidx = jax.lax.broadcasted_iota(jnp.int32, lt.shape, 0)
    m1 = jnp.max(lt, axis=0, keepdims=True)
    i1 = jnp.min(jnp.where(lt == m1, eidx, NUM_EXPERTS), axis=0, keepdims=True)
    masked = jnp.where(eidx == i1, -jnp.inf, lt)
    m2 = jnp.max(masked, axis=0, keepdims=True)
    i2 = jnp.min(jnp.where(masked == m2, eidx, NUM_EXPERTS), axis=0,
                 keepdims=True)
    keep = (eidx == i1) | (eidx == i2)
    scores_ref[...] = jnp.where(keep, jax.nn.sigmoid(lt), 0.0)
    ridx = jax.lax.broadcasted_iota(jnp.int32, (RBLK, HIDDEN), 0)
    idx_ref[...] = (i * RBLK) % TOKENS + ridx


def kernel(hidden_states, W):
    scores, indices = pl.pallas_call(
        _body,
        grid=(GRID,),
        in_specs=[
            pl.BlockSpec((NUM_EXPERTS, HIDDEN), lambda i: (0, 0)),
            pl.BlockSpec((TBLK, HIDDEN), lambda i: (i, 0)),
        ],
        out_specs=[
            pl.BlockSpec((NUM_EXPERTS, TBLK), lambda i: (0, i)),
            pl.BlockSpec((RBLK, HIDDEN), lambda i: (i, 0)),
        ],
        out_shape=[
            jax.ShapeDtypeStruct((NUM_EXPERTS, TOKENS), jnp.float32),
            jax.ShapeDtypeStruct((ROWS, HIDDEN), jnp.int32),
        ],
    )(W, hidden_states)
    probs = scores.reshape(-1, 1)
    return (scores, indices, probs)
